# baseline (speedup 1.0000x reference)
import jax
import jax.numpy as jnp
from jax import lax
from jax.experimental import pallas as pl
from jax.experimental.pallas import tpu as pltpu
from jax.experimental.pallas import tpu_sc as plsc

ROWS = 16384
COLS = 2048
NC = 2
NS = 16
L = 16
NW = NC * NS
RW = ROWS // NW
R = 16
NBLK = RW // R


def _permute_body(x_hbm, perm_hbm, out_hbm, idx_v, in_v, out_v):
    cid = lax.axis_index("c")
    sid = lax.axis_index("s")
    wid = sid * NC + cid
    base = wid * RW * COLS

    pltpu.sync_copy(perm_hbm, idx_v)

    def blk_body(b, _):
        off = base + b * (R * COLS)
        pltpu.sync_copy(x_hbm.at[pl.ds(off, R * COLS)], in_v)

        def col_body(j, _):
            cols = idx_v[pl.ds(j * L, L)]
            for r in range(R):
                vals = plsc.load_gather(in_v, [cols + (r * COLS)])
                out_v[pl.ds(r * COLS + j * L, L)] = vals
            return 0

        lax.fori_loop(0, COLS // L, col_body, 0)
        pltpu.sync_copy(out_v, out_hbm.at[pl.ds(off, R * COLS)])
        return 0

    lax.fori_loop(0, NBLK, blk_body, 0)


def kernel(x, perm, perm_inv):
    del perm_inv
    perm_i32 = perm.astype(jnp.int32)
    x_flat = x.reshape(-1)
    mesh = plsc.VectorSubcoreMesh(core_axis_name="c", subcore_axis_name="s",
                                  num_cores=NC, num_subcores=NS)
    out = pl.kernel(
        _permute_body,
        out_type=jax.ShapeDtypeStruct((ROWS * COLS,), jnp.float32),
        mesh=mesh,
        scratch_types=[
            pltpu.VMEM((COLS,), jnp.int32),
            pltpu.VMEM((R * COLS,), jnp.float32),
            pltpu.VMEM((R * COLS,), jnp.float32),
        ],
        compiler_params=pltpu.CompilerParams(needs_layout_passes=False),
    )(x_flat, perm_i32)
    return (out.reshape(ROWS, COLS), 0)
